# baseline (speedup 1.0000x reference)
import functools

import jax
import jax.numpy as jnp
from jax import lax
from jax.experimental import pallas as pl
from jax.experimental.pallas import tpu as pltpu
from jax.experimental.pallas import tpu_sc as plsc

_NC = 2
_NS = 16
_NW = _NC * _NS
_CH = 16
_NBUF = 2


def kernel(x):
    rows, cols = x.shape
    rows_per_w = rows // _NW
    nch = rows_per_w // _CH
    mesh = plsc.VectorSubcoreMesh(core_axis_name="c", subcore_axis_name="s")

    @functools.partial(
        pl.kernel,
        mesh=mesh,
        out_type=jax.ShapeDtypeStruct((rows, cols), x.dtype),
        scratch_types=[
            pltpu.VMEM((_NBUF, _CH, cols), x.dtype),
            pltpu.SemaphoreType.DMA((_NBUF,)),
            pltpu.SemaphoreType.DMA((_NBUF,)),
        ],
    )
    def k(x_hbm, o_hbm, buf, in_sem, out_sem):
        wid = lax.axis_index("s") * _NC + lax.axis_index("c")
        base = wid * rows_per_w

        def in_copy(i):
            return pltpu.make_async_copy(
                x_hbm.at[pl.ds(base + i * _CH, _CH), :],
                buf.at[i % _NBUF],
                in_sem.at[i % _NBUF],
            )

        def out_copy(i):
            return pltpu.make_async_copy(
                buf.at[i % _NBUF],
                o_hbm.at[pl.ds(base + i * _CH, _CH), :],
                out_sem.at[i % _NBUF],
            )

        for i in range(min(_NBUF, nch)):
            in_copy(i).start()
        for i in range(nch):
            in_copy(i).wait()
            out_copy(i).start()
            if i + _NBUF < nch:
                out_copy(i).wait()
                in_copy(i + _NBUF).start()
        for i in range(max(0, nch - _NBUF), nch):
            out_copy(i).wait()

    return k(x)
